# baseline (speedup 1.0000x reference)
import functools

import jax
import jax.numpy as jnp
from jax import lax
from jax.experimental import pallas as pl
from jax.experimental.pallas import tpu as pltpu
from jax.experimental.pallas import tpu_sc as plsc

D = 64
SCALE = 8.0
NC = 2
NS = 16
NW = NC * NS
CHUNK = 128


def _make_emb(B: int):
    chunks_per_tile = B // (NW * CHUNK)
    mesh = plsc.VectorSubcoreMesh(core_axis_name="c", subcore_axis_name="s")

    @functools.partial(
        pl.kernel,
        mesh=mesh,
        out_type=jax.ShapeDtypeStruct((B, D), jnp.float32),
        scratch_types=[
            pltpu.VMEM((chunks_per_tile, CHUNK), jnp.int32),
            pltpu.VMEM((CHUNK, D), jnp.float32),
            pltpu.SemaphoreType.DMA,
        ],
        compiler_params=pltpu.CompilerParams(use_tc_tiling_on_sc=False),
    )
    def emb(x_hbm, table_hbm, out_hbm, idx_v, rows_v, sem):
        wid = lax.axis_index("s") * NC + lax.axis_index("c")
        pltpu.sync_copy(
            x_hbm.at[pl.ds(wid * chunks_per_tile, chunks_per_tile)], idx_v
        )
        base_row = wid * (chunks_per_tile * CHUNK)

        def chunk_body(j, carry):
            pltpu.async_copy(table_hbm.at[idx_v.at[j]], rows_v, sem).wait()

            def scale_body(i, c2):
                for kk in range(D // 16):
                    sl = pl.ds(kk * 16, 16)
                    rows_v[i, sl] = rows_v[i, sl] * SCALE
                return c2

            lax.fori_loop(0, CHUNK, scale_body, 0)
            pltpu.sync_copy(
                rows_v, out_hbm.at[pl.ds(base_row + j * CHUNK, CHUNK)]
            )
            return carry

        lax.fori_loop(0, chunks_per_tile, chunk_body, 0)

    return emb


def kernel(x, table):
    B = x.shape[0] * x.shape[1]
    xf = x.reshape(B // CHUNK, CHUNK).astype(jnp.int32)
    out = _make_emb(B)(xf, table)
    return out.reshape(x.shape[0], x.shape[1], D)


# speedup vs baseline: 1.2050x; 1.2050x over previous
import functools

import jax
import jax.numpy as jnp
from jax import lax
from jax.experimental import pallas as pl
from jax.experimental.pallas import tpu as pltpu
from jax.experimental.pallas import tpu_sc as plsc

D = 64
SCALE = 8.0
NC = 2
NS = 16
NW = NC * NS
IVEC = 128
CHUNK = 256
NBUF = 4
LOOKAHEAD = 2


def _make_emb(B: int):
    idx_rows = B // (NW * IVEC)
    nslots = B // (NW * CHUNK)
    mesh = plsc.VectorSubcoreMesh(core_axis_name="c", subcore_axis_name="s")

    @functools.partial(
        pl.kernel,
        mesh=mesh,
        out_type=jax.ShapeDtypeStruct((B, D), jnp.float32),
        scratch_types=[
            pltpu.VMEM((idx_rows, IVEC), jnp.int32),
            pltpu.VMEM((NBUF, CHUNK, D), jnp.float32),
            pltpu.SemaphoreType.DMA((NBUF,)),
            pltpu.SemaphoreType.DMA((NBUF,)),
        ],
        compiler_params=pltpu.CompilerParams(use_tc_tiling_on_sc=False),
    )
    def emb(x_hbm, table_hbm, out_hbm, idx_v, rows_v, gsem, ssem):
        wid = lax.axis_index("s") * NC + lax.axis_index("c")
        pltpu.sync_copy(x_hbm.at[pl.ds(wid * idx_rows, idx_rows)], idx_v)
        base_row = wid * (nslots * CHUNK)

        def start_gather(g, b):
            for h in range(CHUNK // IVEC):
                pltpu.async_copy(
                    table_hbm.at[idx_v.at[g * (CHUNK // IVEC) + h]],
                    rows_v.at[b].at[pl.ds(h * IVEC, IVEC)],
                    gsem.at[b],
                )

        def wait_gather(b):
            for h in range(CHUNK // IVEC):
                pltpu.make_async_copy(
                    out_hbm.at[pl.ds(0, IVEC)],
                    rows_v.at[b].at[pl.ds(h * IVEC, IVEC)],
                    gsem.at[b],
                ).wait()

        def start_scatter(g, b):
            pltpu.async_copy(
                rows_v.at[b],
                out_hbm.at[pl.ds(base_row + g * CHUNK, CHUNK)],
                ssem.at[b],
            )

        def wait_scatter(b):
            pltpu.make_async_copy(
                out_hbm.at[pl.ds(0, CHUNK)], rows_v.at[b], ssem.at[b]
            ).wait()

        def scale(b):
            buf = rows_v.at[b]
            rows_per_it = 8

            def body(i, c):
                r0 = i * rows_per_it
                for r in range(rows_per_it):
                    for k in range(D // 16):
                        sl = (r0 + r, pl.ds(k * 16, 16))
                        buf[sl] = buf[sl] * SCALE
                return c

            lax.fori_loop(0, CHUNK // rows_per_it, body, 0)

        for g in range(LOOKAHEAD):
            start_gather(g, g % NBUF)

        def slot(g, carry):
            b = lax.rem(g, NBUF)

            def per_buf(bb):
                @pl.when(b == bb)
                def _():
                    wait_gather(bb)
                    scale(bb)
                    start_scatter(g, bb)

                b2 = (bb + LOOKAHEAD) % NBUF

                @pl.when((b == bb) & (g + LOOKAHEAD < nslots))
                def _():
                    @pl.when(g >= NBUF - LOOKAHEAD)
                    def _():
                        wait_scatter(b2)

                    start_gather(g + LOOKAHEAD, b2)

            for bb in range(NBUF):
                per_buf(bb)
            return carry

        lax.fori_loop(0, nslots, slot, 0)

        for gg in range(nslots - NBUF, nslots):
            wait_scatter(gg % NBUF)

    return emb


def kernel(x, table):
    B = x.shape[0] * x.shape[1]
    xf = x.reshape(B // IVEC, IVEC).astype(jnp.int32)
    out = _make_emb(B)(xf, table)
    return out.reshape(x.shape[0], x.shape[1], D)
